# Initial kernel scaffold; baseline (speedup 1.0000x reference)
import jax
import jax.numpy as jnp
from jax.experimental import pallas as pl


def kernel(h_router, h_packet, W_p, b_p, W_c, b_c, gamma_r, beta_r, gamma_p, beta_p, skip_r, skip_p, eo_src, eo_dst, ei_src, ei_dst, ep_src, ep_dst):
    raise NotImplementedError("write your pallas kernel here")


# speedup vs baseline: 14.2406x; 14.2406x over previous
import functools

import jax
import jax.numpy as jnp
import numpy as np
from jax import lax
from jax.experimental import pallas as pl
from jax.experimental.pallas import tpu as pltpu
from jax.experimental.pallas import tpu_sc as plsc

N_R = 10000
N_C = 20000
N_P = 10000
E_O = 40000
E_I = 40000
E_P = 20000
H = 64
K = H // 2

_NS = 16
_NCORE = 2

_R_NP = np.zeros((H, H * K), np.float32)
for _h in range(H):
    _R_NP[_h, _h * K:(_h + 1) * K] = 1.0
_S_NP = np.zeros((H * K, K), np.float32)
for _h in range(H):
    for _k in range(K):
        _S_NP[_h * K + _k, _k] = 1.0


def _make_segsum(d, n_out, nch):
    n_pad = (n_out + 128) // 128 * 128
    rpt = n_pad // _NS
    nfull, rem = rpt // 128, rpt % 128
    mesh = plsc.VectorSubcoreMesh(core_axis_name="c", subcore_axis_name="s")

    @functools.partial(
        pl.kernel,
        mesh=mesh,
        out_type=jax.ShapeDtypeStruct((_NCORE, n_pad, d), jnp.float32),
        scratch_types=[
            pltpu.VMEM_SHARED((n_pad, d), jnp.float32),
            pltpu.VMEM((nch, 128), jnp.int32),
            pltpu.VMEM((nch, 128), jnp.int32),
            pltpu.VMEM((128, d), jnp.float32),
            pltpu.VMEM((128, d), jnp.float32),
            pltpu.SemaphoreType.DMA,
            pltpu.SemaphoreType.DMA,
        ],
        compiler_params=pltpu.CompilerParams(use_tc_tiling_on_sc=False),
    )
    def seg_kernel(table, srci, dsti, out, acc, siv, div, rowa, rowb, sema, semb):
        c = lax.axis_index("c")
        s = lax.axis_index("s")

        def zrow(i, carry):
            for t in range(d // 16):
                rowa[i, pl.ds(t * 16, 16)] = jnp.zeros((16,), jnp.float32)
            return carry

        lax.fori_loop(0, 128, zrow, 0)
        zbase = s * rpt
        for i in range(nfull):
            pltpu.sync_copy(rowa, acc.at[pl.ds(zbase + i * 128, 128)])
        if rem:
            pltpu.sync_copy(rowa.at[pl.ds(0, rem)],
                            acc.at[pl.ds(zbase + nfull * 128, rem)])

        w = c * _NS + s
        pltpu.sync_copy(srci.at[w], siv)
        pltpu.sync_copy(dsti.at[w], div)
        plsc.subcore_barrier()

        bufs = (rowa, rowb)
        sems = (sema, semb)
        descs = [None, None]
        descs[0] = pltpu.async_copy(table.at[siv.at[0]], rowa, sema)
        for j in range(nch):
            nj = j + 1
            if nj < nch:
                descs[nj % 2] = pltpu.async_copy(
                    table.at[siv.at[nj]], bufs[nj % 2], sems[nj % 2])
            descs[j % 2].wait()
            pltpu.sync_copy(bufs[j % 2], acc.at[div.at[j]], add=True)
        plsc.subcore_barrier()

        for i in range(nfull):
            pltpu.sync_copy(acc.at[pl.ds(zbase + i * 128, 128)],
                            out.at[c, pl.ds(zbase + i * 128, 128)])
        if rem:
            pltpu.sync_copy(acc.at[pl.ds(zbase + nfull * 128, rem)],
                            out.at[c, pl.ds(zbase + nfull * 128, rem)])

    return seg_kernel


def _pad_edges(src, dst, n_pad_to, trash):
    pad = n_pad_to - src.shape[0]
    src = jnp.concatenate([src.astype(jnp.int32),
                           jnp.zeros((pad,), jnp.int32)])
    dst = jnp.concatenate([dst.astype(jnp.int32),
                           jnp.full((pad,), trash, jnp.int32)])
    return src.reshape(-1, 128), dst.reshape(-1, 128)


_BLK1 = 400
_BLK2 = 400


def _tc1_body(hin, hout, u0, u1, wpa, r, s, m_ref, cca_ref):
    u = u0[...] + u1[...]
    a = jnp.dot(u, wpa[...], preferred_element_type=jnp.float32)
    hin_e = jnp.dot(hin[...], r[...], preferred_element_type=jnp.float32)
    hout_e = jnp.dot(hout[...], r[...], preferred_element_type=jnp.float32)
    m_ref[0] = jnp.dot(a * hin_e, s[...], preferred_element_type=jnp.float32)
    m_ref[1] = jnp.dot(a * hout_e, s[...], preferred_element_type=jnp.float32)
    pad16 = (lax.broadcasted_iota(jnp.int32, (_BLK1, 16), 1) == 0)
    cca_ref[...] = jnp.concatenate(
        [hin[...], hout[...], pad16.astype(jnp.float32)], axis=-1)


def _tc1(h_in, h_out, u0, u1, wpa, r, s):
    nblk = N_C // _BLK1
    row = lambda b: (b, 0)
    full = lambda b: (0, 0)
    return pl.pallas_call(
        _tc1_body,
        grid=(nblk,),
        in_specs=[
            pl.BlockSpec((_BLK1, H), row),
            pl.BlockSpec((_BLK1, H), row),
            pl.BlockSpec((_BLK1, 80), row),
            pl.BlockSpec((_BLK1, 80), row),
            pl.BlockSpec((80, H * K), full),
            pl.BlockSpec((H, H * K), full),
            pl.BlockSpec((H * K, K), full),
        ],
        out_specs=[
            pl.BlockSpec((2, _BLK1, K), lambda b: (0, b, 0)),
            pl.BlockSpec((_BLK1, 144), row),
        ],
        out_shape=[
            jax.ShapeDtypeStruct((2, N_C, K), jnp.float32),
            jax.ShapeDtypeStruct((N_C, 144), jnp.float32),
        ],
    )(h_in, h_out, u0, u1, wpa, r, s)


def _layer_norm(x, g, b):
    mu = jnp.mean(x, axis=-1, keepdims=True)
    var = jnp.mean((x - mu) ** 2, axis=-1, keepdims=True)
    return (x - mu) / jnp.sqrt(var + 1e-5) * g + b


def _tc2_body(minr, moutr, hr0, cs0, cs1, hp0, wc, bc, gr, br, gp, bp2,
              sr, sp, m_ref, hr_ref, hp_ref):
    m = jnp.maximum(jnp.concatenate([minr[...], moutr[...]], axis=-1), 0.0)
    m_ref[...] = m
    ar = jax.nn.sigmoid(sr[...])
    hr_ref[...] = _layer_norm(m * ar + hr0[...] * (1.0 - ar), gr[...], br[...])
    cs = cs0[...] + cs1[...]
    cnt = jnp.maximum(cs[:, 128:129], 1.0)
    cp = cs[:, 0:128] / cnt
    c = jnp.maximum(
        jnp.dot(cp, wc[...], preferred_element_type=jnp.float32) + bc[...], 0.0)
    ap = jax.nn.sigmoid(sp[...])
    hp_ref[...] = _layer_norm(c * ap + hp0[...] * (1.0 - ap), gp[...], bp2[...])


def _tc2(minr, moutr, h_router, cs0, cs1, h_packet, wc, bc, gr, br, gp, bp2,
         sr, sp):
    nblk = N_R // _BLK2
    row = lambda b: (b, 0)
    full = lambda b: (0, 0)
    return pl.pallas_call(
        _tc2_body,
        grid=(nblk,),
        in_specs=[
            pl.BlockSpec((_BLK2, K), row),
            pl.BlockSpec((_BLK2, K), row),
            pl.BlockSpec((_BLK2, H), row),
            pl.BlockSpec((_BLK2, 144), row),
            pl.BlockSpec((_BLK2, 144), row),
            pl.BlockSpec((_BLK2, H), row),
            pl.BlockSpec((2 * H, H), full),
            pl.BlockSpec((1, H), full),
            pl.BlockSpec((1, H), full),
            pl.BlockSpec((1, H), full),
            pl.BlockSpec((1, H), full),
            pl.BlockSpec((1, H), full),
            pl.BlockSpec((1, 1), full),
            pl.BlockSpec((1, 1), full),
        ],
        out_specs=[
            pl.BlockSpec((_BLK2, H), row),
            pl.BlockSpec((_BLK2, H), row),
            pl.BlockSpec((_BLK2, H), row),
        ],
        out_shape=[
            jax.ShapeDtypeStruct((N_R, H), jnp.float32),
            jax.ShapeDtypeStruct((N_R, H), jnp.float32),
            jax.ShapeDtypeStruct((N_P, H), jnp.float32),
        ],
    )(minr, moutr, h_router, cs0, cs1, h_packet, wc, bc, gr, br, gp, bp2,
      sr, sp)


_SEG_CACHE = {}


def _segsum(key, **kw):
    if key not in _SEG_CACHE:
        _SEG_CACHE[key] = _make_segsum(**kw)
    return _SEG_CACHE[key]


def kernel(h_router, h_packet, W_p, b_p, W_c, b_c, gamma_r, beta_r,
           gamma_p, beta_p, skip_r, skip_p,
           eo_src, eo_dst, ei_src, ei_dst, ep_src, ep_dst):
    w3 = lambda x: x.reshape(2 * _NS, -1, 128)
    sa0, da0 = _pad_edges(eo_src, eo_dst, 40960, N_C)
    sa1, da1 = _pad_edges(ei_src, ei_dst, 40960, N_C)
    src_a = w3(jnp.concatenate([sa0, sa1]))
    dst_a = w3(jnp.concatenate([da0, da1]))
    src_b, dst_b = _pad_edges(ep_src, ep_dst, 20480, N_C)
    src_b, dst_b = w3(src_b), w3(dst_b)
    sc0, dc0 = _pad_edges(ei_dst, ei_src, 40960, N_R)
    sc1, dc1 = _pad_edges(eo_dst + N_C, eo_src, 40960, N_R)
    src_c = w3(jnp.concatenate([sc0, sc1]))
    dst_c = w3(jnp.concatenate([dc0, dc1]))
    src_d, dst_d = _pad_edges(ep_dst, ep_src, 20480, N_P)
    src_d, dst_d = w3(src_d), w3(dst_d)

    hp_aug = jnp.concatenate(
        [h_packet, jnp.ones((N_P, 1), jnp.float32),
         jnp.zeros((N_P, 15), jnp.float32)], axis=1)
    wpa = jnp.concatenate(
        [W_p, b_p[None, :], jnp.zeros((15, H * K), jnp.float32)], axis=0)
    r_mat = jnp.asarray(_R_NP)
    s_mat = jnp.asarray(_S_NP)

    out_a = _segsum("a", d=H, n_out=N_C, nch=20)(h_router, src_a, dst_a)
    h_in = out_a[0, :N_C]
    h_out = out_a[1, :N_C]
    out_b = _segsum("b", d=80, n_out=N_C, nch=5)(hp_aug, src_b, dst_b)
    m_c, cca = _tc1(h_in, h_out, out_b[0, :N_C], out_b[1, :N_C],
                    wpa, r_mat, s_mat)
    out_c = _segsum("c", d=K, n_out=N_R, nch=20)(m_c.reshape(2 * N_C, K), src_c, dst_c)
    out_d = _segsum("d", d=144, n_out=N_P, nch=5)(cca, src_d, dst_d)
    m, h_r, h_p = _tc2(
        out_c[0, :N_R], out_c[1, :N_R], h_router,
        out_d[0, :N_P], out_d[1, :N_P], h_packet,
        W_c, b_c.reshape(1, H),
        gamma_r.reshape(1, H), beta_r.reshape(1, H),
        gamma_p.reshape(1, H), beta_p.reshape(1, H),
        skip_r.reshape(1, 1), skip_p.reshape(1, 1))
    return (m, h_r, h_p)


# speedup vs baseline: 14.8776x; 1.0447x over previous
import functools

import jax
import jax.numpy as jnp
import numpy as np
from jax import lax
from jax.experimental import pallas as pl
from jax.experimental.pallas import tpu as pltpu
from jax.experimental.pallas import tpu_sc as plsc

N_R = 10000
N_C = 20000
N_P = 10000
E_O = 40000
E_I = 40000
E_P = 20000
H = 64
K = H // 2

_NS = 16
_NCORE = 2
_NW = _NS * _NCORE

_R_NP = np.zeros((H, H * K), np.float32)
for _h in range(H):
    _R_NP[_h, _h * K:(_h + 1) * K] = 1.0
_S_NP = np.zeros((H * K, K), np.float32)
for _h in range(H):
    for _k in range(K):
        _S_NP[_h * K + _k, _k] = 1.0

_CH = 64
_NCH_A = 40
_NCH_B = 10
_STR_A = 40
_STR_B = 16
_ROWS_A = _NW * _STR_A
_ROWS_B = _NW * _STR_B
_BASE_A_SRC = 0
_BASE_A_DST = _BASE_A_SRC + _ROWS_A
_BASE_B_SRC = _BASE_A_DST + _ROWS_A
_BASE_B_DST = _BASE_B_SRC + _ROWS_B
_BASE_C_SRC = _BASE_B_DST + _ROWS_B
_BASE_C_DST = _BASE_C_SRC + _ROWS_A
_BASE_D_SRC = _BASE_C_DST + _ROWS_A
_BASE_D_DST = _BASE_D_SRC + _ROWS_B
_EDGE_ROWS = _BASE_D_DST + _ROWS_B


def _make_segsum(d, n_out, nch, stride, src_base, dst_base, ring, ahead):
    n_pad = (n_out + 128) // 128 * 128
    rpt = n_pad // _NS
    nfull, rem = rpt // _CH, rpt % _CH
    mesh = plsc.VectorSubcoreMesh(core_axis_name="c", subcore_axis_name="s")

    @functools.partial(
        pl.kernel,
        mesh=mesh,
        out_type=jax.ShapeDtypeStruct((_NCORE, n_pad, d), jnp.float32),
        scratch_types=[
            pltpu.VMEM_SHARED((n_pad, d), jnp.float32),
            pltpu.VMEM((nch, _CH), jnp.int32),
            pltpu.VMEM((nch, _CH), jnp.int32),
            pltpu.VMEM((_CH, d), jnp.float32),
            [pltpu.VMEM((_CH, d), jnp.float32) for _ in range(ring)],
            pltpu.SemaphoreType.DMA,
            pltpu.SemaphoreType.DMA,
        ],
        compiler_params=pltpu.CompilerParams(use_tc_tiling_on_sc=False),
    )
    def seg_kernel(table, edges, out, acc, siv, div, zbuf, bufs, gsem, ssem):
        c = lax.axis_index("c")
        s = lax.axis_index("s")
        w = c * _NS + s

        pltpu.sync_copy(edges.at[pl.ds(src_base + w * stride, nch)], siv)
        pltpu.sync_copy(edges.at[pl.ds(dst_base + w * stride, nch)], div)
        gd = [None] * nch
        sd = [None] * nch
        for j in range(min(ahead, nch)):
            gd[j] = pltpu.async_copy(table.at[siv.at[j]], bufs[j % ring], gsem)

        def zrow(i, carry):
            for t in range(d // 16):
                zbuf[i, pl.ds(t * 16, 16)] = jnp.zeros((16,), jnp.float32)
            return carry

        lax.fori_loop(0, _CH, zrow, 0)
        zbase = s * rpt
        for i in range(nfull):
            pltpu.sync_copy(zbuf, acc.at[pl.ds(zbase + i * _CH, _CH)])
        if rem:
            pltpu.sync_copy(zbuf.at[pl.ds(0, rem)],
                            acc.at[pl.ds(zbase + nfull * _CH, rem)])
        plsc.subcore_barrier()

        for j in range(nch):
            if j >= ahead:
                sd[j - ahead].wait()
            nj = j + ahead
            if nj < nch:
                gd[nj] = pltpu.async_copy(
                    table.at[siv.at[nj]], bufs[nj % ring], gsem)
            gd[j].wait()
            sd[j] = pltpu.async_copy(
                bufs[j % ring], acc.at[div.at[j]], ssem, add=True)
        for j in range(max(0, nch - ahead), nch):
            sd[j].wait()
        plsc.subcore_barrier()

        for i in range(nfull):
            pltpu.sync_copy(acc.at[pl.ds(zbase + i * _CH, _CH)],
                            out.at[c, pl.ds(zbase + i * _CH, _CH)])
        if rem:
            pltpu.sync_copy(acc.at[pl.ds(zbase + nfull * _CH, rem)],
                            out.at[c, pl.ds(zbase + nfull * _CH, rem)])

    return seg_kernel


_SEG_CACHE = {}


def _segsum(key, **kw):
    if key not in _SEG_CACHE:
        _SEG_CACHE[key] = _make_segsum(**kw)
    return _SEG_CACHE[key]


def _wm(x, pad_val, n_edges, nch, stride):
    total = _NW * nch * _CH
    x = jnp.concatenate([x.astype(jnp.int32),
                         jnp.full((total - n_edges,), pad_val, jnp.int32)])
    x = x.reshape(_NW, nch, _CH)
    x = jnp.pad(x, ((0, 0), (0, stride - nch), (0, 0)))
    return x.reshape(_NW * stride, _CH)


_BLK1 = 400
_BLK2 = 400


def _tc1_body(hin, hout, u0, u1, wpa, r, s, m_ref, cca_ref):
    u = u0[0] + u1[0]
    a = jnp.dot(u, wpa[...], preferred_element_type=jnp.float32)
    hin_e = jnp.dot(hin[0], r[...], preferred_element_type=jnp.float32)
    hout_e = jnp.dot(hout[0], r[...], preferred_element_type=jnp.float32)
    m_ref[0] = jnp.dot(a * hin_e, s[...], preferred_element_type=jnp.float32)
    m_ref[1] = jnp.dot(a * hout_e, s[...], preferred_element_type=jnp.float32)
    pad16 = (lax.broadcasted_iota(jnp.int32, (_BLK1, 16), 1) == 0)
    cca_ref[...] = jnp.concatenate(
        [hin[0], hout[0], pad16.astype(jnp.float32)], axis=-1)


def _tc1(out_a, out_b, wpa, r, s):
    nblk = N_C // _BLK1
    npad_c = out_a.shape[1]
    lo = lambda b: (0, b, 0)
    hi = lambda b: (1, b, 0)
    full = lambda b: (0, 0)
    return pl.pallas_call(
        _tc1_body,
        grid=(nblk,),
        in_specs=[
            pl.BlockSpec((1, _BLK1, H), lo),
            pl.BlockSpec((1, _BLK1, H), hi),
            pl.BlockSpec((1, _BLK1, 80), lo),
            pl.BlockSpec((1, _BLK1, 80), hi),
            pl.BlockSpec((80, H * K), full),
            pl.BlockSpec((H, H * K), full),
            pl.BlockSpec((H * K, K), full),
        ],
        out_specs=[
            pl.BlockSpec((2, _BLK1, K), lambda b: (0, b, 0)),
            pl.BlockSpec((_BLK1, 144), lambda b: (b, 0)),
        ],
        out_shape=[
            jax.ShapeDtypeStruct((2, N_C, K), jnp.float32),
            jax.ShapeDtypeStruct((N_C, 144), jnp.float32),
        ],
    )(out_a, out_a, out_b, out_b, wpa, r, s)


def _layer_norm(x, g, b):
    mu = jnp.mean(x, axis=-1, keepdims=True)
    var = jnp.mean((x - mu) ** 2, axis=-1, keepdims=True)
    return (x - mu) / jnp.sqrt(var + 1e-5) * g + b


def _tc2_body(minr, moutr, hr0, cs0, cs1, hp0, wc, bc, gr, br, gp, bp2,
              sr, sp, m_ref, hr_ref, hp_ref):
    m = jnp.maximum(jnp.concatenate([minr[0], moutr[0]], axis=-1), 0.0)
    m_ref[...] = m
    ar = jax.nn.sigmoid(sr[...])
    hr_ref[...] = _layer_norm(m * ar + hr0[...] * (1.0 - ar), gr[...], br[...])
    cs = cs0[0] + cs1[0]
    cnt = jnp.maximum(cs[:, 128:129], 1.0)
    cp = cs[:, 0:128] / cnt
    c = jnp.maximum(
        jnp.dot(cp, wc[...], preferred_element_type=jnp.float32) + bc[...], 0.0)
    ap = jax.nn.sigmoid(sp[...])
    hp_ref[...] = _layer_norm(c * ap + hp0[...] * (1.0 - ap), gp[...], bp2[...])


def _tc2(out_c, out_d, h_router, h_packet, wc, bc, gr, br, gp, bp2, sr, sp):
    nblk = N_R // _BLK2
    row = lambda b: (b, 0)
    lo = lambda b: (0, b, 0)
    hi = lambda b: (1, b, 0)
    full = lambda b: (0, 0)
    return pl.pallas_call(
        _tc2_body,
        grid=(nblk,),
        in_specs=[
            pl.BlockSpec((1, _BLK2, K), lo),
            pl.BlockSpec((1, _BLK2, K), hi),
            pl.BlockSpec((_BLK2, H), row),
            pl.BlockSpec((1, _BLK2, 144), lo),
            pl.BlockSpec((1, _BLK2, 144), hi),
            pl.BlockSpec((_BLK2, H), row),
            pl.BlockSpec((2 * H, H), full),
            pl.BlockSpec((1, H), full),
            pl.BlockSpec((1, H), full),
            pl.BlockSpec((1, H), full),
            pl.BlockSpec((1, H), full),
            pl.BlockSpec((1, H), full),
            pl.BlockSpec((1, 1), full),
            pl.BlockSpec((1, 1), full),
        ],
        out_specs=[
            pl.BlockSpec((_BLK2, H), row),
            pl.BlockSpec((_BLK2, H), row),
            pl.BlockSpec((_BLK2, H), row),
        ],
        out_shape=[
            jax.ShapeDtypeStruct((N_R, H), jnp.float32),
            jax.ShapeDtypeStruct((N_R, H), jnp.float32),
            jax.ShapeDtypeStruct((N_P, H), jnp.float32),
        ],
    )(out_c, out_c, h_router, out_d, out_d, h_packet, wc, bc, gr, br, gp,
      bp2, sr, sp)


def kernel(h_router, h_packet, W_p, b_p, W_c, b_c, gamma_r, beta_r,
           gamma_p, beta_p, skip_r, skip_p,
           eo_src, eo_dst, ei_src, ei_dst, ep_src, ep_dst):
    def halves(x0, n0, pv0, x1, n1, pv1, nch, stride):
        total_half = _NS * nch * _CH
        a0 = jnp.concatenate([x0.astype(jnp.int32),
                              jnp.full((total_half - n0,), pv0, jnp.int32)])
        a1 = jnp.concatenate([x1.astype(jnp.int32),
                              jnp.full((total_half - n1,), pv1, jnp.int32)])
        x = jnp.concatenate([a0, a1]).reshape(_NW, nch, _CH)
        x = jnp.pad(x, ((0, 0), (0, stride - nch), (0, 0)))
        return x.reshape(_NW * stride, _CH)

    edges = jnp.concatenate([
        halves(eo_src, E_O, 0, ei_src, E_I, 0, _NCH_A, _STR_A),
        halves(eo_dst, E_O, N_C, ei_dst, E_I, N_C, _NCH_A, _STR_A),
        _wm(ep_src, 0, E_P, _NCH_B, _STR_B),
        _wm(ep_dst, N_C, E_P, _NCH_B, _STR_B),
        halves(ei_dst, E_I, 0, eo_dst + N_C, E_O, 0, _NCH_A, _STR_A),
        halves(ei_src, E_I, N_R, eo_src, E_O, N_R, _NCH_A, _STR_A),
        _wm(ep_dst, 0, E_P, _NCH_B, _STR_B),
        _wm(ep_src, N_P, E_P, _NCH_B, _STR_B),
    ], axis=0)

    hp_aug = jnp.concatenate(
        [h_packet, jnp.ones((N_P, 1), jnp.float32),
         jnp.zeros((N_P, 15), jnp.float32)], axis=1)
    wpa = jnp.concatenate(
        [W_p, b_p[None, :], jnp.zeros((15, H * K), jnp.float32)], axis=0)
    r_mat = jnp.asarray(_R_NP)
    s_mat = jnp.asarray(_S_NP)

    out_a = _segsum("a", d=H, n_out=N_C, nch=_NCH_A, stride=_STR_A,
                    src_base=_BASE_A_SRC, dst_base=_BASE_A_DST,
                    ring=6, ahead=3)(h_router, edges)
    out_b = _segsum("b", d=80, n_out=N_C, nch=_NCH_B, stride=_STR_B,
                    src_base=_BASE_B_SRC, dst_base=_BASE_B_DST,
                    ring=3, ahead=1)(hp_aug, edges)
    m_c, cca = _tc1(out_a, out_b, wpa, r_mat, s_mat)
    out_c = _segsum("c", d=K, n_out=N_R, nch=_NCH_A, stride=_STR_A,
                    src_base=_BASE_C_SRC, dst_base=_BASE_C_DST,
                    ring=8, ahead=4)(
                        m_c.reshape(2 * N_C, K), edges)
    out_d = _segsum("d", d=144, n_out=N_P, nch=_NCH_B, stride=_STR_B,
                    src_base=_BASE_D_SRC, dst_base=_BASE_D_DST,
                    ring=2, ahead=1)(cca, edges)
    m, h_r, h_p = _tc2(
        out_c, out_d, h_router, h_packet,
        W_c, b_c.reshape(1, H),
        gamma_r.reshape(1, H), beta_r.reshape(1, H),
        gamma_p.reshape(1, H), beta_p.reshape(1, H),
        skip_r.reshape(1, 1), skip_p.reshape(1, 1))
    return (m, h_r, h_p)


# speedup vs baseline: 16.2806x; 1.0943x over previous
import functools

import jax
import jax.numpy as jnp
import numpy as np
from jax import lax
from jax.experimental import pallas as pl
from jax.experimental.pallas import tpu as pltpu
from jax.experimental.pallas import tpu_sc as plsc

N_R = 10000
N_C = 20000
N_P = 10000
E_O = 40000
E_I = 40000
E_P = 20000
H = 64
K = H // 2

_NS = 16
_NCORE = 2
_NW = _NS * _NCORE

_R_NP = np.zeros((H, H * K), np.float32)
for _h in range(H):
    _R_NP[_h, _h * K:(_h + 1) * K] = 1.0
_S_NP = np.zeros((H * K, K), np.float32)
for _h in range(H):
    for _k in range(K):
        _S_NP[_h * K + _k, _k] = 1.0

_CH = 64
_NCH_A = 40
_NCH_B = 10
_NCH_D = 20
_STR_A = 40
_STR_B = 16
_STR_D = 24

_NPAD_C = (N_C + 128) // 128 * 128
_NPAD_R = (N_R + 128) // 128 * 128
_NPAD_P = (N_P + 128) // 128 * 128

_MESH = functools.partial(
    plsc.VectorSubcoreMesh, core_axis_name="c", subcore_axis_name="s")


def _zero_acc(zbuf, acc, s, rpt, width):
    nfull, rem = rpt // _CH, rpt % _CH
    zbase = s * rpt
    for i in range(nfull):
        pltpu.sync_copy(zbuf.at[pl.ds(0, _CH), pl.ds(0, width)],
                        acc.at[pl.ds(zbase + i * _CH, _CH)])
    if rem:
        pltpu.sync_copy(zbuf.at[pl.ds(0, rem), pl.ds(0, width)],
                        acc.at[pl.ds(zbase + nfull * _CH, rem)])


def _dump_acc(acc, out, s, rpt):
    nfull, rem = rpt // _CH, rpt % _CH
    zbase = s * rpt
    for i in range(nfull):
        pltpu.sync_copy(acc.at[pl.ds(zbase + i * _CH, _CH)],
                        out.at[pl.ds(zbase + i * _CH, _CH)])
    if rem:
        pltpu.sync_copy(acc.at[pl.ds(zbase + nfull * _CH, rem)],
                        out.at[pl.ds(zbase + nfull * _CH, rem)])


def _make_segsum(d, n_pad, nch, stride, ring, ahead):
    rpt = n_pad // _NS

    @functools.partial(
        pl.kernel,
        mesh=_MESH(),
        out_type=jax.ShapeDtypeStruct((_NCORE, n_pad, d), jnp.float32),
        scratch_types=[
            pltpu.VMEM_SHARED((n_pad, d), jnp.float32),
            pltpu.VMEM((nch, _CH), jnp.int32),
            pltpu.VMEM((nch, _CH), jnp.int32),
            pltpu.VMEM((_CH, d), jnp.float32),
            [pltpu.VMEM((_CH, d), jnp.float32) for _ in range(ring)],
            pltpu.SemaphoreType.DMA,
            pltpu.SemaphoreType.DMA,
        ],
        compiler_params=pltpu.CompilerParams(use_tc_tiling_on_sc=False),
    )
    def seg_kernel(table, edges, out, acc, siv, div, zbuf, bufs, gsem, ssem):
        c = lax.axis_index("c")
        s = lax.axis_index("s")
        w = c * _NS + s
        pltpu.sync_copy(edges.at[pl.ds(w * stride, nch)], siv)
        pltpu.sync_copy(edges.at[pl.ds((_NW + w) * stride, nch)], div)
        gd = [None] * nch
        for j in range(min(ahead, nch)):
            gd[j] = pltpu.async_copy(table.at[siv.at[j]], bufs[j % ring],
                                     gsem)

        def zrow(i, carry):
            for t in range(d // 16):
                zbuf[i, pl.ds(t * 16, 16)] = jnp.zeros((16,), jnp.float32)
            return carry

        lax.fori_loop(0, _CH, zrow, 0)
        _zero_acc(zbuf, acc, s, rpt, d)
        plsc.subcore_barrier()
        sd = [None] * nch
        for j in range(nch):
            if j >= ahead:
                sd[j - ahead].wait()
            nj = j + ahead
            if nj < nch:
                gd[nj] = pltpu.async_copy(table.at[siv.at[nj]],
                                          bufs[nj % ring], gsem)
            gd[j].wait()
            sd[j] = pltpu.async_copy(bufs[j % ring], acc.at[div.at[j]],
                                     ssem, add=True)
        for j in range(max(0, nch - ahead), nch):
            sd[j].wait()
        plsc.subcore_barrier()
        _dump_acc(acc, out.at[c], s, rpt)

    return seg_kernel


def _make_segsum_a(ring, ahead):
    d = H
    nch, stride = _NCH_A, _STR_A
    rpt = _NPAD_C // _NS

    @functools.partial(
        pl.kernel,
        mesh=_MESH(),
        out_type=jax.ShapeDtypeStruct((_NPAD_C, 2 * H), jnp.float32),
        scratch_types=[
            pltpu.VMEM_SHARED((_NPAD_C, d), jnp.float32),
            pltpu.VMEM((nch, _CH), jnp.int32),
            pltpu.VMEM((nch, _CH), jnp.int32),
            pltpu.VMEM((_CH, d), jnp.float32),
            [pltpu.VMEM((_CH, d), jnp.float32) for _ in range(ring)],
            pltpu.SemaphoreType.DMA,
            pltpu.SemaphoreType.DMA,
        ],
        compiler_params=pltpu.CompilerParams(use_tc_tiling_on_sc=False),
    )
    def seg_kernel(table, edges, out, acc, siv, div, zbuf, bufs, gsem, ssem):
        c = lax.axis_index("c")
        s = lax.axis_index("s")
        w = c * _NS + s
        pltpu.sync_copy(edges.at[pl.ds(w * stride, nch)], siv)
        pltpu.sync_copy(edges.at[pl.ds((_NW + w) * stride, nch)], div)
        gd = [None] * nch
        for j in range(min(ahead, nch)):
            gd[j] = pltpu.async_copy(table.at[siv.at[j]], bufs[j % ring],
                                     gsem)

        def zrow(i, carry):
            for t in range(d // 16):
                zbuf[i, pl.ds(t * 16, 16)] = jnp.zeros((16,), jnp.float32)
            return carry

        lax.fori_loop(0, _CH, zrow, 0)
        _zero_acc(zbuf, acc, s, rpt, d)
        plsc.subcore_barrier()
        sd = [None] * nch
        for j in range(nch):
            if j >= ahead:
                sd[j - ahead].wait()
            nj = j + ahead
            if nj < nch:
                gd[nj] = pltpu.async_copy(table.at[siv.at[nj]],
                                          bufs[nj % ring], gsem)
            gd[j].wait()
            sd[j] = pltpu.async_copy(bufs[j % ring], acc.at[div.at[j]],
                                     ssem, add=True)
        for j in range(max(0, nch - ahead), nch):
            sd[j].wait()
        plsc.subcore_barrier()
        nfull, rem = rpt // _CH, rpt % _CH
        zbase = s * rpt
        for i in range(nfull):
            pltpu.sync_copy(acc.at[pl.ds(zbase + i * _CH, _CH)],
                            out.at[pl.ds(zbase + i * _CH, _CH),
                                   pl.ds(c * H, H)])
        if rem:
            pltpu.sync_copy(acc.at[pl.ds(zbase + nfull * _CH, rem)],
                            out.at[pl.ds(zbase + nfull * _CH, rem),
                                   pl.ds(c * H, H)])

    return seg_kernel


def _make_segsum_cnt(n_pad, nch, stride, ring, ahead):
    d = H
    rpt = n_pad // _NS

    @functools.partial(
        pl.kernel,
        mesh=_MESH(),
        out_type=[
            jax.ShapeDtypeStruct((_NCORE, n_pad, d), jnp.float32),
            jax.ShapeDtypeStruct((_NCORE, n_pad, 16), jnp.float32),
        ],
        scratch_types=[
            pltpu.VMEM_SHARED((n_pad, d), jnp.float32),
            pltpu.VMEM_SHARED((n_pad, 16), jnp.float32),
            pltpu.VMEM((nch, _CH), jnp.int32),
            pltpu.VMEM((nch, _CH), jnp.int32),
            pltpu.VMEM((_CH, d), jnp.float32),
            pltpu.VMEM((_CH, 16), jnp.float32),
            [pltpu.VMEM((_CH, d), jnp.float32) for _ in range(ring)],
            pltpu.SemaphoreType.DMA,
            pltpu.SemaphoreType.DMA,
        ],
        compiler_params=pltpu.CompilerParams(use_tc_tiling_on_sc=False),
    )
    def seg_kernel(table, edges, out, outn, acc, accn, siv, div, zbuf, ones,
                   bufs, gsem, ssem):
        c = lax.axis_index("c")
        s = lax.axis_index("s")
        w = c * _NS + s
        pltpu.sync_copy(edges.at[pl.ds(w * stride, nch)], siv)
        pltpu.sync_copy(edges.at[pl.ds((_NW + w) * stride, nch)], div)
        gd = [None] * nch
        for j in range(min(ahead, nch)):
            gd[j] = pltpu.async_copy(table.at[siv.at[j]], bufs[j % ring],
                                     gsem)

        def zrow(i, carry):
            for t in range(d // 16):
                zbuf[i, pl.ds(t * 16, 16)] = jnp.zeros((16,), jnp.float32)
            ones[i, pl.ds(0, 16)] = jnp.ones((16,), jnp.float32)
            return carry

        lax.fori_loop(0, _CH, zrow, 0)
        _zero_acc(zbuf, acc, s, rpt, d)
        _zero_acc(zbuf, accn, s, rpt, 16)
        plsc.subcore_barrier()
        sd = [None] * nch
        sn = [None] * nch
        for j in range(nch):
            if j >= ahead:
                sd[j - ahead].wait()
                sn[j - ahead].wait()
            nj = j + ahead
            if nj < nch:
                gd[nj] = pltpu.async_copy(table.at[siv.at[nj]],
                                          bufs[nj % ring], gsem)
            gd[j].wait()
            sd[j] = pltpu.async_copy(bufs[j % ring], acc.at[div.at[j]],
                                     ssem, add=True)
            sn[j] = pltpu.async_copy(ones, accn.at[div.at[j]], ssem,
                                     add=True)
        for j in range(max(0, nch - ahead), nch):
            sd[j].wait()
            sn[j].wait()
        plsc.subcore_barrier()
        _dump_acc(acc, out.at[c], s, rpt)
        _dump_acc(accn, outn.at[c], s, rpt)

    return seg_kernel


_SEG_CACHE = {}


def _seg(key, maker, *a, **kw):
    if key not in _SEG_CACHE:
        _SEG_CACHE[key] = maker(*a, **kw)
    return _SEG_CACHE[key]


def _wm_pair(src, pv_s, dst, pv_d, n_edges, nch, stride):
    total = _NW * nch * _CH
    pieces = []
    for x, pv in ((src, pv_s), (dst, pv_d)):
        x = jnp.concatenate([x.astype(jnp.int32),
                             jnp.full((total - n_edges,), pv, jnp.int32)])
        x = x.reshape(_NW, nch, _CH)
        x = jnp.pad(x, ((0, 0), (0, stride - nch), (0, 0)))
        pieces.append(x.reshape(_NW * stride, _CH))
    return jnp.concatenate(pieces, axis=0)


def _wm_halves(s0, pv_s0, s1, pv_s1, d0, pv_d0, d1, pv_d1, n0, n1, nch,
               stride):
    half = _NS * nch * _CH
    out = []
    for x, pv, n in ((s0, pv_s0, n0), (s1, pv_s1, n1),
                     (d0, pv_d0, n0), (d1, pv_d1, n1)):
        x = jnp.concatenate([x.astype(jnp.int32),
                             jnp.full((half - n,), pv, jnp.int32)])
        x = x.reshape(_NS, nch, _CH)
        if stride != nch:
            x = jnp.pad(x, ((0, 0), (0, stride - nch), (0, 0)))
        out.append(x.reshape(_NS * stride, _CH))
    return jnp.concatenate(out, axis=0)


_BLK1 = 400
_BLK2 = 400


def _tc1_body(ha, u0, u1, n0, n1, wpa, r, s, m_ref):
    ua = jnp.concatenate([u0[0] + u1[0], n0[0] + n1[0]],
                         axis=-1).astype(jnp.bfloat16)
    a = jnp.dot(ua, wpa[...], preferred_element_type=jnp.float32)
    hin = ha[:, 0:H].astype(jnp.bfloat16)
    hout = ha[:, H:2 * H].astype(jnp.bfloat16)
    hin_e = jnp.dot(hin, r[...], preferred_element_type=jnp.float32)
    hout_e = jnp.dot(hout, r[...], preferred_element_type=jnp.float32)
    m_ref[0] = jnp.dot((a * hin_e).astype(jnp.bfloat16), s[...],
                       preferred_element_type=jnp.float32)
    m_ref[1] = jnp.dot((a * hout_e).astype(jnp.bfloat16), s[...],
                       preferred_element_type=jnp.float32)


def _tc1(out_a, out_bu, out_bn, wpa, r, s):
    nblk = N_C // _BLK1
    lo = lambda b: (0, b, 0)
    hi = lambda b: (1, b, 0)
    full = lambda b: (0, 0)
    return pl.pallas_call(
        _tc1_body,
        grid=(nblk,),
        in_specs=[
            pl.BlockSpec((_BLK1, 2 * H), lambda b: (b, 0)),
            pl.BlockSpec((1, _BLK1, H), lo),
            pl.BlockSpec((1, _BLK1, H), hi),
            pl.BlockSpec((1, _BLK1, 16), lo),
            pl.BlockSpec((1, _BLK1, 16), hi),
            pl.BlockSpec((80, H * K), full),
            pl.BlockSpec((H, H * K), full),
            pl.BlockSpec((H * K, K), full),
        ],
        out_specs=pl.BlockSpec((2, _BLK1, K), lambda b: (0, b, 0)),
        out_shape=jax.ShapeDtypeStruct((2, N_C, K), jnp.float32),
    )(out_a, out_bu, out_bu, out_bn, out_bn, wpa, r, s)


def _layer_norm(x, g, b):
    mu = jnp.mean(x, axis=-1, keepdims=True)
    var = jnp.mean((x - mu) ** 2, axis=-1, keepdims=True)
    return (x - mu) / jnp.sqrt(var + 1e-5) * g + b


def _tc2_body(minr, moutr, hr0, cs0, cs1, cn0, hp0, wc, bc, gr, br,
              gp, bp2, sr, sp, m_ref, hr_ref, hp_ref):
    m = jnp.maximum(jnp.concatenate([minr[0], moutr[0]], axis=-1), 0.0)
    m_ref[...] = m
    ar = jax.nn.sigmoid(sr[...])
    hr_ref[...] = _layer_norm(m * ar + hr0[...] * (1.0 - ar), gr[...], br[...])
    cs = jnp.concatenate([cs0[0], cs1[0]], axis=-1)
    cnt = jnp.maximum(cn0[0, :, 0:1], 1.0)
    cp = cs / cnt
    c = jnp.maximum(
        jnp.dot(cp, wc[...], preferred_element_type=jnp.float32) + bc[...], 0.0)
    ap = jax.nn.sigmoid(sp[...])
    hp_ref[...] = _layer_norm(c * ap + hp0[...] * (1.0 - ap), gp[...], bp2[...])


def _tc2(out_c, out_d, out_dn, h_router, h_packet, wc, bc, gr, br, gp, bp2,
         sr, sp):
    nblk = N_R // _BLK2
    row = lambda b: (b, 0)
    lo = lambda b: (0, b, 0)
    hi = lambda b: (1, b, 0)
    full = lambda b: (0, 0)
    return pl.pallas_call(
        _tc2_body,
        grid=(nblk,),
        in_specs=[
            pl.BlockSpec((1, _BLK2, K), lo),
            pl.BlockSpec((1, _BLK2, K), hi),
            pl.BlockSpec((_BLK2, H), row),
            pl.BlockSpec((1, _BLK2, H), lo),
            pl.BlockSpec((1, _BLK2, H), hi),
            pl.BlockSpec((1, _BLK2, 16), lo),
            pl.BlockSpec((_BLK2, H), row),
            pl.BlockSpec((2 * H, H), full),
            pl.BlockSpec((1, H), full),
            pl.BlockSpec((1, H), full),
            pl.BlockSpec((1, H), full),
            pl.BlockSpec((1, H), full),
            pl.BlockSpec((1, H), full),
            pl.BlockSpec((1, 1), full),
            pl.BlockSpec((1, 1), full),
        ],
        out_specs=[
            pl.BlockSpec((_BLK2, H), row),
            pl.BlockSpec((_BLK2, H), row),
            pl.BlockSpec((_BLK2, H), row),
        ],
        out_shape=[
            jax.ShapeDtypeStruct((N_R, H), jnp.float32),
            jax.ShapeDtypeStruct((N_R, H), jnp.float32),
            jax.ShapeDtypeStruct((N_P, H), jnp.float32),
        ],
    )(out_c, out_c, h_router, out_d, out_d, out_dn, h_packet, wc,
      bc, gr, br, gp, bp2, sr, sp)


def kernel(h_router, h_packet, W_p, b_p, W_c, b_c, gamma_r, beta_r,
           gamma_p, beta_p, skip_r, skip_p,
           eo_src, eo_dst, ei_src, ei_dst, ep_src, ep_dst):
    edges_b = _wm_pair(ep_src, 0, ep_dst, N_C, E_P, _NCH_B, _STR_B)
    edges_a = _wm_halves(eo_src, 0, ei_src, 0, eo_dst, N_C, ei_dst, N_C,
                         E_O, E_I, _NCH_A, _STR_A)
    ep_dst32 = ep_dst.astype(jnp.int32)
    edges_d = _wm_halves(2 * ep_dst32, 0, 2 * ep_dst32 + 1, 0,
                         ep_src, N_P, ep_src, N_P, E_P, E_P,
                         _NCH_D, _STR_D)
    edges_c = _wm_halves(ei_dst, 0, eo_dst + N_C, 0, ei_src, N_R,
                         eo_src, N_R, E_I, E_O, _NCH_A, _STR_A)

    wpa = jnp.concatenate(
        [W_p, b_p[None, :], jnp.zeros((15, H * K), jnp.float32)],
        axis=0).astype(jnp.bfloat16)
    r_mat = jnp.asarray(_R_NP).astype(jnp.bfloat16)
    s_mat = jnp.asarray(_S_NP).astype(jnp.bfloat16)

    out_bu, out_bn = _seg("b", _make_segsum_cnt, n_pad=_NPAD_C, nch=_NCH_B,
                          stride=_STR_B, ring=5, ahead=2)(h_packet, edges_b)
    out_a = _seg("a", _make_segsum_a, ring=8, ahead=4)(h_router, edges_a)
    out_d, out_dn = _seg("d", _make_segsum_cnt, n_pad=_NPAD_P, nch=_NCH_D,
                         stride=_STR_D, ring=8, ahead=4)(
                             out_a.reshape(2 * _NPAD_C, H), edges_d)
    m_c = _tc1(out_a, out_bu, out_bn, wpa, r_mat, s_mat)
    out_c = _seg("c", _make_segsum, d=K, n_pad=_NPAD_R, nch=_NCH_A,
                 stride=_STR_A, ring=8, ahead=4)(
                     m_c.reshape(2 * N_C, K), edges_c)
    m, h_r, h_p = _tc2(
        out_c, out_d, out_dn, h_router, h_packet,
        W_c, b_c.reshape(1, H),
        gamma_r.reshape(1, H), beta_r.reshape(1, H),
        gamma_p.reshape(1, H), beta_p.reshape(1, H),
        skip_r.reshape(1, 1), skip_p.reshape(1, 1))
    return (m, h_r, h_p)


# speedup vs baseline: 16.2927x; 1.0007x over previous
import functools

import jax
import jax.numpy as jnp
import numpy as np
from jax import lax
from jax.experimental import pallas as pl
from jax.experimental.pallas import tpu as pltpu
from jax.experimental.pallas import tpu_sc as plsc

N_R = 10000
N_C = 20000
N_P = 10000
E_O = 40000
E_I = 40000
E_P = 20000
H = 64
K = H // 2

_NS = 16
_NCORE = 2
_NW = _NS * _NCORE

_R_NP = np.zeros((H, H * K), np.float32)
for _h in range(H):
    _R_NP[_h, _h * K:(_h + 1) * K] = 1.0
_S_NP = np.zeros((H * K, K), np.float32)
for _h in range(H):
    for _k in range(K):
        _S_NP[_h * K + _k, _k] = 1.0

_CH = 64
_NCH_A = 40
_NCH_B = 10
_NCH_D = 20
_STR_A = 40
_STR_B = 16
_STR_D = 24

_NPAD_C = (N_C + 128) // 128 * 128
_NPAD_R = (N_R + 128) // 128 * 128
_NPAD_P = (N_P + 128) // 128 * 128

_MESH = functools.partial(
    plsc.VectorSubcoreMesh, core_axis_name="c", subcore_axis_name="s")


def _zero_acc(zbuf, acc, s, rpt, width):
    nfull, rem = rpt // _CH, rpt % _CH
    zbase = s * rpt
    for i in range(nfull):
        pltpu.sync_copy(zbuf.at[pl.ds(0, _CH), pl.ds(0, width)],
                        acc.at[pl.ds(zbase + i * _CH, _CH)])
    if rem:
        pltpu.sync_copy(zbuf.at[pl.ds(0, rem), pl.ds(0, width)],
                        acc.at[pl.ds(zbase + nfull * _CH, rem)])


def _dump_acc(acc, out, s, rpt):
    nfull, rem = rpt // _CH, rpt % _CH
    zbase = s * rpt
    for i in range(nfull):
        pltpu.sync_copy(acc.at[pl.ds(zbase + i * _CH, _CH)],
                        out.at[pl.ds(zbase + i * _CH, _CH)])
    if rem:
        pltpu.sync_copy(acc.at[pl.ds(zbase + nfull * _CH, rem)],
                        out.at[pl.ds(zbase + nfull * _CH, rem)])


def _make_segsum(d, n_pad, nch, stride, ring, ahead):
    rpt = n_pad // _NS

    @functools.partial(
        pl.kernel,
        mesh=_MESH(),
        out_type=jax.ShapeDtypeStruct((_NCORE, n_pad, d), jnp.float32),
        scratch_types=[
            pltpu.VMEM_SHARED((n_pad, d), jnp.float32),
            pltpu.VMEM((nch, _CH), jnp.int32),
            pltpu.VMEM((nch, _CH), jnp.int32),
            pltpu.VMEM((_CH, d), jnp.float32),
            [pltpu.VMEM((_CH, d), jnp.float32) for _ in range(ring)],
            pltpu.SemaphoreType.DMA,
            pltpu.SemaphoreType.DMA,
        ],
        compiler_params=pltpu.CompilerParams(use_tc_tiling_on_sc=False),
    )
    def seg_kernel(table, edges, out, acc, siv, div, zbuf, bufs, gsem, ssem):
        c = lax.axis_index("c")
        s = lax.axis_index("s")
        w = c * _NS + s
        pltpu.sync_copy(edges.at[pl.ds(w * stride, nch)], siv)
        pltpu.sync_copy(edges.at[pl.ds((_NW + w) * stride, nch)], div)
        gd = [None] * nch
        for j in range(min(ahead, nch)):
            gd[j] = pltpu.async_copy(table.at[siv.at[j]], bufs[j % ring],
                                     gsem)

        def zrow(i, carry):
            for t in range(d // 16):
                zbuf[i, pl.ds(t * 16, 16)] = jnp.zeros((16,), jnp.float32)
            return carry

        lax.fori_loop(0, _CH, zrow, 0)
        _zero_acc(zbuf, acc, s, rpt, d)
        plsc.subcore_barrier()
        sd = [None] * nch
        for j in range(nch):
            if j >= ahead:
                sd[j - ahead].wait()
            nj = j + ahead
            if nj < nch:
                gd[nj] = pltpu.async_copy(table.at[siv.at[nj]],
                                          bufs[nj % ring], gsem)
            gd[j].wait()
            sd[j] = pltpu.async_copy(bufs[j % ring], acc.at[div.at[j]],
                                     ssem, add=True)
        for j in range(max(0, nch - ahead), nch):
            sd[j].wait()
        plsc.subcore_barrier()
        _dump_acc(acc, out.at[c], s, rpt)

    return seg_kernel


def _make_segsum_a(ring, ahead):
    d = H
    nch, stride = _NCH_A, _STR_A
    rpt = _NPAD_C // _NS

    @functools.partial(
        pl.kernel,
        mesh=_MESH(),
        out_type=jax.ShapeDtypeStruct((_NPAD_C, 2 * H), jnp.float32),
        scratch_types=[
            pltpu.VMEM_SHARED((_NPAD_C, d), jnp.float32),
            pltpu.VMEM((nch, _CH), jnp.int32),
            pltpu.VMEM((nch, _CH), jnp.int32),
            pltpu.VMEM((_CH, d), jnp.float32),
            [pltpu.VMEM((_CH, d), jnp.float32) for _ in range(ring)],
            pltpu.SemaphoreType.DMA,
            pltpu.SemaphoreType.DMA,
        ],
        compiler_params=pltpu.CompilerParams(use_tc_tiling_on_sc=False),
    )
    def seg_kernel(table, edges, out, acc, siv, div, zbuf, bufs, gsem, ssem):
        c = lax.axis_index("c")
        s = lax.axis_index("s")
        w = c * _NS + s
        pltpu.sync_copy(edges.at[pl.ds(w * stride, nch)], siv)
        pltpu.sync_copy(edges.at[pl.ds((_NW + w) * stride, nch)], div)
        gd = [None] * nch
        for j in range(min(ahead, nch)):
            gd[j] = pltpu.async_copy(table.at[siv.at[j]], bufs[j % ring],
                                     gsem)

        def zrow(i, carry):
            for t in range(d // 16):
                zbuf[i, pl.ds(t * 16, 16)] = jnp.zeros((16,), jnp.float32)
            return carry

        lax.fori_loop(0, _CH, zrow, 0)
        _zero_acc(zbuf, acc, s, rpt, d)
        plsc.subcore_barrier()
        sd = [None] * nch
        for j in range(nch):
            if j >= ahead:
                sd[j - ahead].wait()
            nj = j + ahead
            if nj < nch:
                gd[nj] = pltpu.async_copy(table.at[siv.at[nj]],
                                          bufs[nj % ring], gsem)
            gd[j].wait()
            sd[j] = pltpu.async_copy(bufs[j % ring], acc.at[div.at[j]],
                                     ssem, add=True)
        for j in range(max(0, nch - ahead), nch):
            sd[j].wait()
        plsc.subcore_barrier()
        nfull, rem = rpt // _CH, rpt % _CH
        zbase = s * rpt
        for i in range(nfull):
            pltpu.sync_copy(acc.at[pl.ds(zbase + i * _CH, _CH)],
                            out.at[pl.ds(zbase + i * _CH, _CH),
                                   pl.ds(c * H, H)])
        if rem:
            pltpu.sync_copy(acc.at[pl.ds(zbase + nfull * _CH, rem)],
                            out.at[pl.ds(zbase + nfull * _CH, rem),
                                   pl.ds(c * H, H)])

    return seg_kernel


def _make_segsum_cnt(d, n_pad, nch, stride, ring, ahead):
    rpt = n_pad // _NS

    @functools.partial(
        pl.kernel,
        mesh=_MESH(),
        out_type=[
            jax.ShapeDtypeStruct((_NCORE, n_pad, d), jnp.float32),
            jax.ShapeDtypeStruct((_NCORE, n_pad, 16), jnp.float32),
        ],
        scratch_types=[
            pltpu.VMEM_SHARED((n_pad, d), jnp.float32),
            pltpu.VMEM_SHARED((n_pad, 16), jnp.float32),
            pltpu.VMEM((nch, _CH), jnp.int32),
            pltpu.VMEM((nch, _CH), jnp.int32),
            pltpu.VMEM((_CH, d), jnp.float32),
            pltpu.VMEM((_CH, 16), jnp.float32),
            [pltpu.VMEM((_CH, d), jnp.float32) for _ in range(ring)],
            pltpu.SemaphoreType.DMA,
            pltpu.SemaphoreType.DMA,
        ],
        compiler_params=pltpu.CompilerParams(use_tc_tiling_on_sc=False),
    )
    def seg_kernel(table, edges, out, outn, acc, accn, siv, div, zbuf, ones,
                   bufs, gsem, ssem):
        c = lax.axis_index("c")
        s = lax.axis_index("s")
        w = c * _NS + s
        pltpu.sync_copy(edges.at[pl.ds(w * stride, nch)], siv)
        pltpu.sync_copy(edges.at[pl.ds((_NW + w) * stride, nch)], div)
        gd = [None] * nch
        for j in range(min(ahead, nch)):
            gd[j] = pltpu.async_copy(table.at[siv.at[j]], bufs[j % ring],
                                     gsem)

        def zrow(i, carry):
            for t in range(d // 16):
                zbuf[i, pl.ds(t * 16, 16)] = jnp.zeros((16,), jnp.float32)
            ones[i, pl.ds(0, 16)] = jnp.ones((16,), jnp.float32)
            return carry

        lax.fori_loop(0, _CH, zrow, 0)
        _zero_acc(zbuf, acc, s, rpt, d)
        _zero_acc(zbuf, accn, s, rpt, 16)
        plsc.subcore_barrier()
        sd = [None] * nch
        sn = [None] * nch
        for j in range(nch):
            if j >= ahead:
                sd[j - ahead].wait()
                sn[j - ahead].wait()
            nj = j + ahead
            if nj < nch:
                gd[nj] = pltpu.async_copy(table.at[siv.at[nj]],
                                          bufs[nj % ring], gsem)
            gd[j].wait()
            sd[j] = pltpu.async_copy(bufs[j % ring], acc.at[div.at[j]],
                                     ssem, add=True)
            sn[j] = pltpu.async_copy(ones, accn.at[div.at[j]], ssem,
                                     add=True)
        for j in range(max(0, nch - ahead), nch):
            sd[j].wait()
            sn[j].wait()
        plsc.subcore_barrier()
        _dump_acc(acc, out.at[c], s, rpt)
        _dump_acc(accn, outn.at[c], s, rpt)

    return seg_kernel


_SEG_CACHE = {}


def _seg(key, maker, *a, **kw):
    if key not in _SEG_CACHE:
        _SEG_CACHE[key] = maker(*a, **kw)
    return _SEG_CACHE[key]


def _wm_pair(src, pv_s, dst, pv_d, n_edges, nch, stride):
    total = _NW * nch * _CH
    pieces = []
    for x, pv in ((src, pv_s), (dst, pv_d)):
        x = jnp.concatenate([x.astype(jnp.int32),
                             jnp.full((total - n_edges,), pv, jnp.int32)])
        x = x.reshape(_NW, nch, _CH)
        x = jnp.pad(x, ((0, 0), (0, stride - nch), (0, 0)))
        pieces.append(x.reshape(_NW * stride, _CH))
    return jnp.concatenate(pieces, axis=0)


def _wm_halves(s0, pv_s0, s1, pv_s1, d0, pv_d0, d1, pv_d1, n0, n1, nch,
               stride):
    half = _NS * nch * _CH
    out = []
    for x, pv, n in ((s0, pv_s0, n0), (s1, pv_s1, n1),
                     (d0, pv_d0, n0), (d1, pv_d1, n1)):
        x = jnp.concatenate([x.astype(jnp.int32),
                             jnp.full((half - n,), pv, jnp.int32)])
        x = x.reshape(_NS, nch, _CH)
        if stride != nch:
            x = jnp.pad(x, ((0, 0), (0, stride - nch), (0, 0)))
        out.append(x.reshape(_NS * stride, _CH))
    return jnp.concatenate(out, axis=0)


_BLK1 = 400
_BLK2 = 400


def _tc1_body(ha, u0, u1, wpa, r, s, m_ref):
    ua = u0[0] + u1[0]
    a = jnp.dot(ua, wpa[...], preferred_element_type=jnp.float32)
    hin_e = jnp.dot(ha[:, 0:H], r[...], preferred_element_type=jnp.float32)
    hout_e = jnp.dot(ha[:, H:2 * H], r[...],
                     preferred_element_type=jnp.float32)
    m_ref[0] = jnp.dot(a * hin_e, s[...], preferred_element_type=jnp.float32)
    m_ref[1] = jnp.dot(a * hout_e, s[...], preferred_element_type=jnp.float32)


def _tc1(out_a, out_b, wpa, r, s):
    nblk = N_C // _BLK1
    lo = lambda b: (0, b, 0)
    hi = lambda b: (1, b, 0)
    full = lambda b: (0, 0)
    return pl.pallas_call(
        _tc1_body,
        grid=(nblk,),
        in_specs=[
            pl.BlockSpec((_BLK1, 2 * H), lambda b: (b, 0)),
            pl.BlockSpec((1, _BLK1, 80), lo),
            pl.BlockSpec((1, _BLK1, 80), hi),
            pl.BlockSpec((80, H * K), full),
            pl.BlockSpec((H, H * K), full),
            pl.BlockSpec((H * K, K), full),
        ],
        out_specs=pl.BlockSpec((2, _BLK1, K), lambda b: (0, b, 0)),
        out_shape=jax.ShapeDtypeStruct((2, N_C, K), jnp.float32),
    )(out_a, out_b, out_b, wpa, r, s)


def _layer_norm(x, g, b):
    mu = jnp.mean(x, axis=-1, keepdims=True)
    var = jnp.mean((x - mu) ** 2, axis=-1, keepdims=True)
    return (x - mu) / jnp.sqrt(var + 1e-5) * g + b


def _tc2_body(minr, moutr, hr0, cs0, cs1, cn0, cn1, hp0, wc, bc, gr, br,
              gp, bp2, sr, sp, m_ref, hr_ref, hp_ref):
    m = jnp.maximum(jnp.concatenate([minr[0], moutr[0]], axis=-1), 0.0)
    m_ref[...] = m
    ar = jax.nn.sigmoid(sr[...])
    hr_ref[...] = _layer_norm(m * ar + hr0[...] * (1.0 - ar), gr[...], br[...])
    cs = cs0[0] + cs1[0]
    cnt = jnp.maximum(cn0[0, :, 0:1] + cn1[0, :, 0:1], 1.0)
    cp = cs / cnt
    c = jnp.maximum(
        jnp.dot(cp, wc[...], preferred_element_type=jnp.float32) + bc[...], 0.0)
    ap = jax.nn.sigmoid(sp[...])
    hp_ref[...] = _layer_norm(c * ap + hp0[...] * (1.0 - ap), gp[...], bp2[...])


def _tc2(out_c, out_d, out_dn, h_router, h_packet, wc, bc, gr, br, gp, bp2,
         sr, sp):
    nblk = N_R // _BLK2
    row = lambda b: (b, 0)
    lo = lambda b: (0, b, 0)
    hi = lambda b: (1, b, 0)
    full = lambda b: (0, 0)
    return pl.pallas_call(
        _tc2_body,
        grid=(nblk,),
        in_specs=[
            pl.BlockSpec((1, _BLK2, K), lo),
            pl.BlockSpec((1, _BLK2, K), hi),
            pl.BlockSpec((_BLK2, H), row),
            pl.BlockSpec((1, _BLK2, 2 * H), lo),
            pl.BlockSpec((1, _BLK2, 2 * H), hi),
            pl.BlockSpec((1, _BLK2, 16), lo),
            pl.BlockSpec((1, _BLK2, 16), hi),
            pl.BlockSpec((_BLK2, H), row),
            pl.BlockSpec((2 * H, H), full),
            pl.BlockSpec((1, H), full),
            pl.BlockSpec((1, H), full),
            pl.BlockSpec((1, H), full),
            pl.BlockSpec((1, H), full),
            pl.BlockSpec((1, H), full),
            pl.BlockSpec((1, 1), full),
            pl.BlockSpec((1, 1), full),
        ],
        out_specs=[
            pl.BlockSpec((_BLK2, H), row),
            pl.BlockSpec((_BLK2, H), row),
            pl.BlockSpec((_BLK2, H), row),
        ],
        out_shape=[
            jax.ShapeDtypeStruct((N_R, H), jnp.float32),
            jax.ShapeDtypeStruct((N_R, H), jnp.float32),
            jax.ShapeDtypeStruct((N_P, H), jnp.float32),
        ],
    )(out_c, out_c, h_router, out_d, out_d, out_dn, out_dn, h_packet, wc,
      bc, gr, br, gp, bp2, sr, sp)


def kernel(h_router, h_packet, W_p, b_p, W_c, b_c, gamma_r, beta_r,
           gamma_p, beta_p, skip_r, skip_p,
           eo_src, eo_dst, ei_src, ei_dst, ep_src, ep_dst):
    edges_b = _wm_pair(ep_src, 0, ep_dst, N_C, E_P, _NCH_B, _STR_B)
    edges_a = _wm_halves(eo_src, 0, ei_src, 0, eo_dst, N_C, ei_dst, N_C,
                         E_O, E_I, _NCH_A, _STR_A)
    edges_d = _wm_pair(ep_dst, 0, ep_src, N_P, E_P, _NCH_B, _STR_B)
    edges_c = _wm_halves(ei_dst, 0, eo_dst + N_C, 0, ei_src, N_R,
                         eo_src, N_R, E_I, E_O, _NCH_A, _STR_A)

    hp_aug = jnp.concatenate(
        [h_packet, jnp.ones((N_P, 1), jnp.float32),
         jnp.zeros((N_P, 15), jnp.float32)], axis=1)
    wpa = jnp.concatenate(
        [W_p, b_p[None, :], jnp.zeros((15, H * K), jnp.float32)], axis=0)
    r_mat = jnp.asarray(_R_NP)
    s_mat = jnp.asarray(_S_NP)

    out_b = _seg("b", _make_segsum, d=80, n_pad=_NPAD_C, nch=_NCH_B,
                 stride=_STR_B, ring=4, ahead=2)(hp_aug, edges_b)
    out_a = _seg("a", _make_segsum_a, ring=8, ahead=4)(h_router, edges_a)
    out_d, out_dn = _seg("d", _make_segsum_cnt, d=2 * H, n_pad=_NPAD_P,
                         nch=_NCH_B, stride=_STR_B, ring=3, ahead=2)(
                             out_a, edges_d)
    m_c = _tc1(out_a, out_b, wpa, r_mat, s_mat)
    out_c = _seg("c", _make_segsum, d=K, n_pad=_NPAD_R, nch=_NCH_A,
                 stride=_STR_A, ring=8, ahead=4)(
                     m_c.reshape(2 * N_C, K), edges_c)
    m, h_r, h_p = _tc2(
        out_c, out_d, out_dn, h_router, h_packet,
        W_c, b_c.reshape(1, H),
        gamma_r.reshape(1, H), beta_r.reshape(1, H),
        gamma_p.reshape(1, H), beta_p.reshape(1, H),
        skip_r.reshape(1, 1), skip_p.reshape(1, 1))
    return (m, h_r, h_p)
